# Initial kernel scaffold; baseline (speedup 1.0000x reference)
import jax
import jax.numpy as jnp
from jax.experimental import pallas as pl


def kernel(x, s, Wk):
    raise NotImplementedError("write your pallas kernel here")


# speedup vs baseline: 8.0824x; 8.0824x over previous
import functools

import jax
import jax.numpy as jnp
from jax import lax
from jax.experimental import pallas as pl

T = 8192
D = 256
W = 512
R = 128
NW = T // W
NB = 384
CB = 2048
NCB = T // CB

_DEFAULT = lax.Precision.DEFAULT
_HIGHEST = lax.Precision.HIGHEST


def _plan_body(x_ref, wk_ref, m_ref, selodd_ref):
    xw = x_ref[...]
    wk = wk_ref[...]
    keys = lax.dot_general(xw, wk, (((1,), (0,)), ((), ())),
                           precision=_DEFAULT)
    nrm = jnp.sqrt(jnp.sum(keys * keys, axis=1, keepdims=True)) + 1e-6
    kn = keys / nrm
    sf = lax.dot_general(kn, kn, (((1,), (1,)), ((), ())),
                         precision=_DEFAULT)

    ir = lax.broadcasted_iota(jnp.int32, (W, W), 0)
    ic = lax.broadcasted_iota(jnp.int32, (W, W), 1)
    odd_c = (ic % 2) == 1
    even_c = jnp.logical_not(odd_c)

    masked = jnp.where(odd_c, sf, -3.0)
    bs_col = jnp.max(masked, axis=1, keepdims=True)
    bd_col = jnp.min(jnp.where((sf == bs_col) & odd_c, ic, W),
                     axis=1, keepdims=True)
    bdj_col = bd_col // 2

    ir1 = lax.broadcasted_iota(jnp.int32, (W, 1), 0)
    bs_row = jnp.transpose(bs_col)
    gt = (bs_row > bs_col) & even_c
    eqlt = (bs_row == bs_col) & even_c & (ic < ir)
    rank_col = jnp.sum((gt | eqlt).astype(jnp.float32), axis=1, keepdims=True)

    even_col = (ir1 % 2) == 0
    sel_col = even_col & (rank_col < float(R))
    unm_col = even_col & jnp.logical_not(sel_col)
    unm_f = unm_col.astype(jnp.float32)

    tri = (ic <= ir).astype(jnp.float32)
    pos_col = lax.dot_general(tri, unm_f, (((1,), (0,)), ((), ())),
                              precision=_HIGHEST) - 1.0

    cols = jnp.concatenate(
        [pos_col, sel_col.astype(jnp.float32), unm_f,
         bdj_col.astype(jnp.float32)], axis=1)
    rows = jnp.transpose(cols)
    pos_row = rows[0:1]
    sel_row = rows[1:2] > 0.5
    unm_row = rows[2:3] > 0.5
    bdj_row = rows[3:4]

    r2 = lax.broadcasted_iota(jnp.int32, (NB, W), 0)
    c2 = lax.broadcasted_iota(jnp.int32, (NB, W), 1)
    r2f = r2.astype(jnp.float32)
    cj = c2 // 2
    top = (r2 < R) & unm_row & (pos_row == r2f)
    bot_own = (r2 >= R) & ((c2 % 2) == 1) & (cj == (r2 - R))
    bot_src = (r2 >= R) & sel_row & (bdj_row == (r2f - float(R)))
    m_ref[0] = (top | bot_own | bot_src).astype(jnp.float32)

    selodd_ref[0] = (sel_col | jnp.logical_not(even_col)).astype(jnp.float32)


def _merge_body(m_ref, s_ref, ns_ref, sz_ref):
    cb = pl.program_id(1)
    mw = m_ref[0]
    sw = s_ref[...]
    ns_ref[...] = lax.dot_general(mw, sw, (((1,), (0,)), ((), ())),
                                  precision=_HIGHEST)
    part = jnp.sum(sw, axis=1, keepdims=True)

    @pl.when(cb == 0)
    def _():
        sz_ref[0] = part

    @pl.when(cb != 0)
    def _():
        sz_ref[0] += part


def _xmerge_body(x_ref, m_ref, sz_ref, selodd_ref, nx_ref):
    xw = x_ref[...]
    mw = m_ref[0]
    sz = sz_ref[0]
    v = jnp.where(selodd_ref[0] > 0.5, sz, 1.0)
    num = lax.dot_general(mw, xw * v, (((1,), (0,)), ((), ())),
                          precision=_HIGHEST)
    den = lax.dot_general(mw, v, (((1,), (0,)), ((), ())),
                          precision=_HIGHEST)
    nx_ref[...] = num / jnp.clip(den, 1e-6)[...]


@jax.jit
def kernel(x, s, Wk):
    m, selodd = pl.pallas_call(
        _plan_body,
        grid=(NW,),
        in_specs=[
            pl.BlockSpec((W, D), lambda w: (w, 0)),
            pl.BlockSpec((D, D), lambda w: (0, 0)),
        ],
        out_specs=[
            pl.BlockSpec((1, NB, W), lambda w: (w, 0, 0)),
            pl.BlockSpec((1, W, 1), lambda w: (w, 0, 0)),
        ],
        out_shape=[
            jax.ShapeDtypeStruct((NW, NB, W), jnp.float32),
            jax.ShapeDtypeStruct((NW, W, 1), jnp.float32),
        ],
    )(x, Wk)

    ns, sizes = pl.pallas_call(
        _merge_body,
        grid=(NW, NCB),
        in_specs=[
            pl.BlockSpec((1, NB, W), lambda w, cb: (w, 0, 0)),
            pl.BlockSpec((W, CB), lambda w, cb: (w, cb)),
        ],
        out_specs=[
            pl.BlockSpec((NB, CB), lambda w, cb: (w, cb)),
            pl.BlockSpec((1, W, 1), lambda w, cb: (w, 0, 0)),
        ],
        out_shape=[
            jax.ShapeDtypeStruct((NW * NB, T), jnp.float32),
            jax.ShapeDtypeStruct((NW, W, 1), jnp.float32),
        ],
    )(m, s)

    nx = pl.pallas_call(
        _xmerge_body,
        grid=(NW,),
        in_specs=[
            pl.BlockSpec((W, D), lambda w: (w, 0)),
            pl.BlockSpec((1, NB, W), lambda w: (w, 0, 0)),
            pl.BlockSpec((1, W, 1), lambda w: (w, 0, 0)),
            pl.BlockSpec((1, W, 1), lambda w: (w, 0, 0)),
        ],
        out_specs=pl.BlockSpec((NB, D), lambda w: (w, 0)),
        out_shape=jax.ShapeDtypeStruct((NW * NB, D), jnp.float32),
    )(x, m, sizes, selodd)

    return nx, ns
